# baseline (speedup 1.0000x reference)
import jax
import jax.numpy as jnp
from jax.experimental import pallas as pl

_TOTAL = 32 * 3 * 16 * 128 * 128
_COLS = 16384
_ROWS = _TOTAL // _COLS
_BLK = 128
_GRID = _ROWS // _BLK


def _kl_block(p1_ref, p2_ref, out_ref):
    p1 = p1_ref[...]
    p2 = p2_ref[...]
    eps = 1e-10
    np1 = 1.0 - p1
    np2 = 1.0 - p2
    kl = p1 * jnp.log(p1 / p2 + eps) + np1 * jnp.log(np1 / np2 + eps)
    s = jnp.sum(kl).reshape(1, 1)

    @pl.when(pl.program_id(0) == 0)
    def _init():
        out_ref[...] = s

    @pl.when(pl.program_id(0) != 0)
    def _acc():
        out_ref[...] += s


def kernel(pred1, pred2):
    p1 = pred1.reshape(_ROWS, _COLS)
    p2 = pred2.reshape(_ROWS, _COLS)
    out = pl.pallas_call(
        _kl_block,
        grid=(_GRID,),
        in_specs=[
            pl.BlockSpec((_BLK, _COLS), lambda i: (i, 0)),
            pl.BlockSpec((_BLK, _COLS), lambda i: (i, 0)),
        ],
        out_specs=pl.BlockSpec((1, 1), lambda i: (0, 0)),
        out_shape=jax.ShapeDtypeStruct((1, 1), jnp.float32),
    )(p1, p2)
    return out[0, 0]
